# baseline (speedup 1.0000x reference)
import functools

import jax
import jax.numpy as jnp
from jax import lax
from jax.experimental import pallas as pl
from jax.experimental.pallas import tpu as pltpu
from jax.experimental.pallas import tpu_sc as plsc

_HEADS = 16
_N = 2048
_QROWS = 128
_NQ = _N // _QROWS
_W8 = 4352
_WA = 4096

_NW = 32
_PER_W = 160
_VT = _NW * _PER_W


def _sc_gather_kernel(table_ref, out_ref, idx_ref, rows_ref, sem):
    wid = lax.axis_index("s") * 2 + lax.axis_index("c")
    base = wid * _PER_W
    for c in range(2):
        for jj in range(5):
            m = base + 80 * c + 16 * jj + lax.iota(jnp.int32, 16)
            n = 2303 - m
            ret = jnp.where(n < 0, 16, 0)
            na = jnp.abs(n)
            bits = lax.bitcast_convert_type(na.astype(jnp.float32), jnp.int32)
            e2 = ((bits >> 23) - 127) * 2
            frac_hi = jnp.where((bits & 0x7FFFFF) >= 3474676, 1, 0)
            val_large = jnp.minimum(2 + e2 + frac_hi, 15)
            idx_ref[c, 16 * jj : 16 * (jj + 1)] = ret + jnp.where(
                na < 8, na, val_large
            )
    copies = [
        pltpu.async_copy(table_ref.at[idx_ref.at[c]], rows_ref.at[c], sem)
        for c in range(2)
    ]
    for cp in copies:
        cp.wait()
    for c in range(2):
        pltpu.sync_copy(
            rows_ref.at[c], out_ref.at[pl.ds(base + 80 * c, 80)]
        )


def _sc_gather(table):
    run = pl.kernel(
        _sc_gather_kernel,
        mesh=plsc.VectorSubcoreMesh(core_axis_name="c", subcore_axis_name="s"),
        out_type=jax.ShapeDtypeStruct((_VT, 128), jnp.float32),
        scratch_types=[
            pltpu.VMEM((2, 80), jnp.int32),
            pltpu.VMEM((2, 80, 128), jnp.float32),
            pltpu.SemaphoreType.DMA,
        ],
    )
    return run(table)


def _fill_kernel(vtt_ref, out_ref, v8_ref, va_ref, sem_ref):
    h = pl.program_id(0)
    p = jax.lax.rem(h, 2)

    def _copy(src_parity, head, q):
        start = 1920 - _QROWS * q
        return pltpu.make_async_copy(
            va_ref.at[src_parity, :, start : start + _N],
            out_ref.at[head, pl.ds(_QROWS * q, _QROWS), :],
            sem_ref.at[src_parity],
        )

    @pl.when(h >= 2)
    def _drain_prev():
        for q in range(_NQ):
            _copy(p, h - 2, q).wait()

    for s in range(8):
        v8_ref[s : s + 1, :] = vtt_ref[0, 0:1, 128 - s : 128 - s + _W8]

    for k in range(_QROWS // 8):
        off = 255 - 8 * k
        va_ref[p, 8 * k : 8 * (k + 1), :] = v8_ref[:, off : off + _WA]

    for q in range(_NQ):
        _copy(p, h, q).start()

    @pl.when(h == _HEADS - 1)
    def _drain_tail():
        for q in range(_NQ):
            _copy(1 - p, h - 1, q).wait()
        for q in range(_NQ):
            _copy(p, h, q).wait()


def kernel(n, relative_attention_bias):
    del n
    table128 = jnp.pad(relative_attention_bias, ((0, 0), (0, 112)))
    vt = _sc_gather(table128)
    vtt = jnp.transpose(vt[:, :_HEADS]).reshape(_HEADS, 1, _VT)
    return pl.pallas_call(
        _fill_kernel,
        grid=(_HEADS,),
        in_specs=[
            pl.BlockSpec((1, 1, _VT), lambda h: (h, 0, 0)),
        ],
        out_specs=pl.BlockSpec(memory_space=pl.ANY),
        out_shape=jax.ShapeDtypeStruct((_HEADS, _N, _N), jnp.float32),
        scratch_shapes=[
            pltpu.VMEM((8, _W8), jnp.float32),
            pltpu.VMEM((2, _QROWS, _WA), jnp.float32),
            pltpu.SemaphoreType.DMA((2,)),
        ],
    )(vtt)


# speedup vs baseline: 2.5132x; 2.5132x over previous
import math

import jax
import jax.numpy as jnp
from jax.experimental import pallas as pl
from jax.experimental.pallas import tpu as pltpu

_HEADS = 16
_N = 2048
_QROWS = 128
_NQ = _N // _QROWS
_W8 = 4352
_WA = 4096


def _fill_kernel(table_ref, out_ref, bkt_ref, v8_ref, va_ref, sem_ref):
    h = pl.program_id(0)
    p = jax.lax.rem(h, 2)

    @pl.when(h == 0)
    def _build_bucket():
        s_io = jax.lax.broadcasted_iota(jnp.int32, (8, _W8), 0)
        d_io = jax.lax.broadcasted_iota(jnp.int32, (8, _W8), 1)
        rel = d_io - s_io - 2175
        n = -rel
        ret = jnp.where(n < 0, 16, 0)
        na = jnp.abs(n)
        is_small = na < 8
        naf = jnp.maximum(na, 8).astype(jnp.float32)
        val_large = 8 + (
            jnp.log(naf / 8.0) / math.log(128.0 / 8.0) * 8.0
        ).astype(jnp.int32)
        val_large = jnp.minimum(val_large, 15)
        bkt_ref[...] = ret + jnp.where(is_small, na, val_large)

    def _copy(src_parity, head, q):
        start = 1920 - _QROWS * q
        return pltpu.make_async_copy(
            va_ref.at[src_parity, :, start : start + _N],
            out_ref.at[head, pl.ds(_QROWS * q, _QROWS), :],
            sem_ref.at[src_parity, q % 4],
        )

    @pl.when(h >= 2)
    def _drain_prev():
        for q in range(_NQ):
            _copy(p, h - 2, q).wait()

    bucket = bkt_ref[...]
    v8_ref[...] = jax.lax.fori_loop(
        0,
        32,
        lambda b, a: jnp.where(bucket == b, table_ref[b, h], a),
        jnp.zeros((8, _W8), jnp.float32),
    )

    for k in range(_QROWS // 8):
        off = 255 - 8 * k
        va_ref[p, 8 * k : 8 * (k + 1), :] = v8_ref[:, off : off + _WA]

    for q in range(_NQ):
        _copy(p, h, q).start()

    @pl.when(h == _HEADS - 1)
    def _drain_tail():
        for q in range(_NQ):
            _copy(1 - p, h - 1, q).wait()
        for q in range(_NQ):
            _copy(p, h, q).wait()


def kernel(n, relative_attention_bias):
    del n
    return pl.pallas_call(
        _fill_kernel,
        grid=(_HEADS,),
        in_specs=[
            pl.BlockSpec((32, _HEADS), lambda h: (0, 0),
                         memory_space=pltpu.SMEM),
        ],
        out_specs=pl.BlockSpec(memory_space=pl.ANY),
        out_shape=jax.ShapeDtypeStruct((_HEADS, _N, _N), jnp.float32),
        scratch_shapes=[
            pltpu.VMEM((8, _W8), jnp.int32),
            pltpu.VMEM((8, _W8), jnp.float32),
            pltpu.VMEM((2, _QROWS, _WA), jnp.float32),
            pltpu.SemaphoreType.DMA((2, 4)),
        ],
    )(relative_attention_bias)
